# baseline (speedup 1.0000x reference)
import jax
import jax.numpy as jnp
from jax.experimental import pallas as pl
from jax.experimental.pallas import tpu as pltpu

B = 4
T = 2048
D_PROJ = 1024
D_C = 512
D_MODEL = 1024
N_HEADS = 16
HD = D_MODEL // N_HEADS
S0 = 64
GATE_R = 8
PER_SPK = 32
SQ = 2 * PER_SPK
SIGMA = 4.0

TB = 2048
NT = T // TB

_DNT = (((1,), (1,)), ((), ()))
F32 = jnp.float32
BF16 = jnp.bfloat16


def _dott(a, b):
    return jax.lax.dot_general(a, b, _DNT, preferred_element_type=F32)


def _dot(a, b):
    return jax.lax.dot_general(a, b, (((1,), (0,)), ((), ())),
                               preferred_element_type=F32)


def _spk_track(h_ref, a_ref, sp_ref, wkv_ref, bkv_ref, wq, bq, wqin, bqin):
    a_row = a_ref[0]
    s_row = sp_ref[0]
    s_col = jnp.transpose(s_row)

    t_row = jax.lax.broadcasted_iota(jnp.int32, (S0, T), 1)
    dist = t_row - s_col

    acc = jnp.zeros((S0, 1), F32)
    cnt = jnp.zeros((S0, 1), jnp.int32)
    for off in range(-GATE_R, GATE_R + 1):
        m = dist == off
        tap = jnp.sum(jnp.where(m, a_row, 0.0), axis=1, keepdims=True)
        acc = acc + tap
        idx = s_col + off
        cnt = cnt + ((idx >= 0) & (idx < T)).astype(jnp.int32)
    scores = acc / jnp.maximum(cnt, 1).astype(F32)
    scores_row = jnp.transpose(scores)

    ii = jax.lax.broadcasted_iota(jnp.int32, (S0, S0), 0)
    jj = jax.lax.broadcasted_iota(jnp.int32, (S0, S0), 1)
    gt = (scores_row > scores).astype(jnp.int32)
    eq = ((scores_row == scores) & (jj < ii)).astype(jnp.int32)
    rank = jnp.sum(gt + eq, axis=1, keepdims=True)
    rank_row = jnp.transpose(rank)

    r_col = jax.lax.broadcasted_iota(jnp.int32, (PER_SPK, 1), 0)
    sel = (rank_row == r_col).astype(jnp.int32)
    p = jnp.sum(sel * s_row, axis=1, keepdims=True)
    conf = jnp.sum(sel.astype(F32) * scores_row, axis=1, keepdims=True)
    gate = jax.nn.sigmoid(2.0 * conf)

    t2 = jax.lax.broadcasted_iota(jnp.int32, (PER_SPK, T), 1)
    d2 = t2 - p
    win = (d2 >= -GATE_R) & (d2 <= GATE_R)
    df = d2.astype(F32) * (1.0 / SIGMA)
    w = jnp.where(win, jnp.exp(-0.5 * df * df) * a_row, 0.0)
    wsum = jnp.sum(w, axis=1, keepdims=True)
    wn = (w / (wsum + 1e-6)).astype(BF16)
    z = _dot(wn, h_ref[0]).astype(BF16)

    k_seed = _dott(z, wkv_ref[0]) + bkv_ref[...]
    qk = jnp.tanh(_dott(k_seed.astype(BF16), wq) + bq).astype(BF16)
    return (_dott(qk, wqin) + bqin).astype(BF16), gate


def _prep_kernel(h0_ref, a0_ref, sp0_ref, wkv0_ref, bkv0_ref,
                 h1_ref, a1_ref, sp1_ref, wkv1_ref, bkv1_ref,
                 wq_ref, bq_ref, wqin_ref, bqin_ref, q_out, g_out):
    wq = wq_ref[...]
    bq = bq_ref[...]
    wqin = wqin_ref[...]
    bqin = bqin_ref[...]
    q0, g0 = _spk_track(h0_ref, a0_ref, sp0_ref, wkv0_ref, bkv0_ref,
                        wq, bq, wqin, bqin)
    q1, g1 = _spk_track(h1_ref, a1_ref, sp1_ref, wkv1_ref, bkv1_ref,
                        wq, bq, wqin, bqin)
    q_out[0, 0:PER_SPK] = q0
    q_out[0, PER_SPK:SQ] = q1
    g_out[0, 0:PER_SPK] = g0
    g_out[0, PER_SPK:SQ] = g1


def _prep(h0, a0, sp0, wkv0, bkv0, h1, a1, sp1, wkv1, bkv1, wq, bq, wqin,
          bqin):
    hspec = pl.BlockSpec((1, T, D_C), lambda b: (b, 0, 0))
    aspec = pl.BlockSpec((1, 1, T), lambda b: (b, 0, 0))
    sspec = pl.BlockSpec((1, 1, S0), lambda b: (b, 0, 0))
    wkvspec = pl.BlockSpec((1, D_MODEL, D_C), lambda b: (0, 0, 0))
    rowspec = pl.BlockSpec((1, D_MODEL), lambda b: (0, 0))
    wspec = pl.BlockSpec((D_MODEL, D_MODEL), lambda b: (0, 0))
    return pl.pallas_call(
        _prep_kernel,
        grid=(B,),
        in_specs=[hspec, aspec, sspec, wkvspec, rowspec,
                  hspec, aspec, sspec, wkvspec, rowspec,
                  wspec, rowspec, wspec, rowspec],
        out_specs=[
            pl.BlockSpec((1, SQ, D_MODEL), lambda b: (b, 0, 0)),
            pl.BlockSpec((1, SQ, 1), lambda b: (b, 0, 0)),
        ],
        out_shape=[
            jax.ShapeDtypeStruct((B, SQ, D_MODEL), BF16),
            jax.ShapeDtypeStruct((B, SQ, 1), F32),
        ],
    )(h0, a0, sp0, wkv0, bkv0, h1, a1, sp1, wkv1, bkv1, wq, bq, wqin, bqin)


def _wfuse_kernel(wk_ref, wv_ref, wmem_ref, bmem_ref, bv_ref,
                  wkf_out, wvf_out, bvf_out):
    wmem = wmem_ref[...]
    wkf_out[...] = _dot(wk_ref[...], wmem).astype(BF16)
    wvf_out[...] = _dot(wv_ref[...], wmem).astype(BF16)
    bvf_out[...] = _dott(bmem_ref[...].astype(BF16), wv_ref[...]) + bv_ref[...]


def _wfuse(wk, wv, wmem, bmem, bv):
    def full(shape):
        return pl.BlockSpec(shape, lambda: tuple(0 for _ in shape))
    return pl.pallas_call(
        _wfuse_kernel,
        in_specs=[full((D_MODEL, D_MODEL)), full((D_MODEL, D_MODEL)),
                  full((D_MODEL, D_PROJ)), full((1, D_MODEL)),
                  full((1, D_MODEL))],
        out_specs=[full((D_MODEL, D_PROJ)), full((D_MODEL, D_PROJ)),
                   full((1, D_MODEL))],
        out_shape=[
            jax.ShapeDtypeStruct((D_MODEL, D_PROJ), BF16),
            jax.ShapeDtypeStruct((D_MODEL, D_PROJ), BF16),
            jax.ShapeDtypeStruct((1, D_MODEL), F32),
        ],
    )(wk, wv, wmem, bmem, bv)


def _attn_kernel(pf_ref, wkf_ref, wvf_ref, bvf_ref, q_ref,
                 opw_ref, opb_ref, wo_ref, bo_ref, g_ref, a0_ref, a1_ref,
                 tags_ref, out_ref, m_s, l_s, acc_s, o_s, sc_s, pv_s):
    tb = pl.program_id(1)
    x = pf_ref[0]
    kb = _dott(x, wkf_ref[...]).astype(BF16)
    vb = (_dott(x, wvf_ref[...]) + bvf_ref[...]).astype(BF16)
    qa = q_ref[0]

    for h in range(N_HEADS):
        qh = qa[:, h * HD:(h + 1) * HD]
        kh = kb[:, h * HD:(h + 1) * HD]
        sc_s[h] = _dott(qh, kh) * (1.0 / (HD ** 0.5))

    sc = sc_s[...]
    m_loc = jnp.max(sc, axis=2, keepdims=True)
    if NT > 1:
        m_old = jnp.where(tb == 0, jnp.full((N_HEADS, SQ, 1), -1e30, F32),
                          m_s[...])
        m_new = jnp.maximum(m_old, m_loc)
        resc = jnp.exp(m_old - m_new)
        m_s[...] = m_new
    else:
        m_new = m_loc
    e = jnp.exp(sc - m_new)
    lloc = jnp.sum(e, axis=2, keepdims=True)
    if NT > 1:
        l_old = jnp.where(tb == 0, jnp.zeros((N_HEADS, SQ, 1), F32), l_s[...])
        l_s[...] = l_old * resc + lloc
    eb = e.astype(BF16)

    for h in range(N_HEADS):
        pv_s[h] = _dot(eb[h], vb[:, h * HD:(h + 1) * HD])
    if NT > 1:
        acc_old = jnp.where(tb == 0, jnp.zeros((N_HEADS, SQ, HD), F32),
                            acc_s[...])
        acc_s[...] = acc_old * resc + pv_s[...]
    @pl.when(tb == NT - 1)
    def _():
        if NT > 1:
            onorm = acc_s[...] / l_s[...]
        else:
            onorm = pv_s[...] / lloc
        for h in range(N_HEADS):
            o_s[:, h * HD:(h + 1) * HD] = onorm[h]

    @pl.when(tb == NT - 1)
    def _():
        o = o_s[...]
        f = _dott(o, opw_ref[...]) + opb_ref[...]
        f = _dott(f, wo_ref[...]) + bo_ref[...]
        g = g_ref[0]
        a0 = a0_ref[0, :, 0:1]
        a1 = a1_ref[0, :, 0:1]
        den = a0 + a1 + 1e-6
        tags = tags_ref[...]
        slot = (a0 / den) * tags[0:1, :] + (a1 / den) * tags[1:2, :]
        out_ref[0] = f * g + slot


def _attn_out(pf, wkf, wvf, bvf, q_all, opw, opb, wo, bo, g_all, a0s,
              a1s, tags):
    stride = T // SQ
    wide = pl.BlockSpec((D_MODEL, D_PROJ), lambda b, t: (0, 0))
    row = pl.BlockSpec((1, D_MODEL), lambda b, t: (0, 0))
    sqd = pl.BlockSpec((1, SQ, D_MODEL), lambda b, t: (b, 0, 0))
    return pl.pallas_call(
        _attn_kernel,
        grid=(B, NT),
        in_specs=[
            pl.BlockSpec((1, TB, D_PROJ), lambda b, t: (b, t, 0)),
            wide, wide, row, sqd,
            pl.BlockSpec((D_MODEL, D_MODEL), lambda b, t: (0, 0)), row,
            pl.BlockSpec((D_MODEL, D_MODEL), lambda b, t: (0, 0)), row,
            pl.BlockSpec((1, SQ, 1), lambda b, t: (b, 0, 0)),
            pl.BlockSpec((1, SQ, stride), lambda b, t: (b, 0, 0)),
            pl.BlockSpec((1, SQ, stride), lambda b, t: (b, 0, 0)),
            pl.BlockSpec((2, D_MODEL), lambda b, t: (0, 0)),
        ],
        out_specs=sqd,
        out_shape=jax.ShapeDtypeStruct((B, SQ, D_MODEL), F32),
        scratch_shapes=[
            pltpu.VMEM((N_HEADS, SQ, 1), F32),
            pltpu.VMEM((N_HEADS, SQ, 1), F32),
            pltpu.VMEM((N_HEADS, SQ, HD), F32),
            pltpu.VMEM((SQ, D_MODEL), F32),
            pltpu.VMEM((N_HEADS, SQ, TB), F32),
            pltpu.VMEM((N_HEADS, SQ, HD), F32),
        ],
    )(pf, wkf, wvf, bvf, q_all, opw, opb, wo, bo, g_all, a0s, a1s, tags)


def kernel(proj_feats, h_ctc_0, h_ctc_1, A_0, A_1, spikes_0, spikes_1,
           W_mem, b_mem, W_kv_0, b_kv_0, W_kv_1, b_kv_1, W_q, b_q, W_o, b_o,
           in_proj_w, in_proj_b, out_proj_w, out_proj_b, tags):
    wqi = in_proj_w[0:D_MODEL].astype(BF16)
    wki = in_proj_w[D_MODEL:2 * D_MODEL].astype(BF16)
    wvi = in_proj_w[2 * D_MODEL:3 * D_MODEL].astype(BF16)
    bqi = in_proj_b[0:D_MODEL].reshape(1, D_MODEL)
    bvi = in_proj_b[2 * D_MODEL:3 * D_MODEL].reshape(1, D_MODEL)

    q_all, g_all = _prep(
        h_ctc_0.astype(BF16), A_0.reshape(B, 1, T),
        spikes_0.reshape(B, 1, S0),
        W_kv_0[:D_MODEL].reshape(1, D_MODEL, D_C).astype(BF16),
        b_kv_0[:D_MODEL].reshape(1, D_MODEL),
        h_ctc_1.astype(BF16), A_1.reshape(B, 1, T),
        spikes_1.reshape(B, 1, S0),
        W_kv_1[:D_MODEL].reshape(1, D_MODEL, D_C).astype(BF16),
        b_kv_1[:D_MODEL].reshape(1, D_MODEL),
        W_q.astype(BF16), b_q.reshape(1, D_MODEL), wqi, bqi)

    wkf, wvf, bvf = _wfuse(wki, wvi, W_mem.astype(BF16),
                           b_mem.reshape(1, D_MODEL), bvi)

    a0s = A_0.reshape(B, SQ, T // SQ)
    a1s = A_1.reshape(B, SQ, T // SQ)
    return _attn_out(proj_feats.astype(BF16), wkf, wvf, bvf, q_all,
                     out_proj_w, out_proj_b.reshape(1, D_MODEL), W_o,
                     b_o.reshape(1, D_MODEL), g_all, a0s, a1s, tags)


# speedup vs baseline: 1.2204x; 1.2204x over previous
import jax
import jax.numpy as jnp
from jax.experimental import pallas as pl
from jax.experimental.pallas import tpu as pltpu

B = 4
T = 2048
D_PROJ = 1024
D_C = 512
D_MODEL = 1024
N_HEADS = 16
HD = D_MODEL // N_HEADS
S0 = 64
GATE_R = 8
PER_SPK = 32
SQ = 2 * PER_SPK
SIGMA = 4.0

TB = 2048
NT = T // TB

_DNT = (((1,), (1,)), ((), ()))
F32 = jnp.float32
BF16 = jnp.bfloat16


def _dott(a, b):
    return jax.lax.dot_general(a, b, _DNT, preferred_element_type=F32)


def _dot(a, b):
    return jax.lax.dot_general(a, b, (((1,), (0,)), ((), ())),
                               preferred_element_type=F32)


def _spk_track(h_ref, a_ref, sp_ref, wkv_ref, bkv_ref, wq, bq, wqin, bqin):
    a_row = a_ref[0]
    s_row = sp_ref[0]
    s_col = jnp.transpose(s_row)

    t_row = jax.lax.broadcasted_iota(jnp.int32, (S0, T), 1)
    dist = t_row - s_col

    acc = jnp.zeros((S0, 1), F32)
    cnt = jnp.zeros((S0, 1), jnp.int32)
    for off in range(-GATE_R, GATE_R + 1):
        m = dist == off
        tap = jnp.sum(jnp.where(m, a_row, 0.0), axis=1, keepdims=True)
        acc = acc + tap
        idx = s_col + off
        cnt = cnt + ((idx >= 0) & (idx < T)).astype(jnp.int32)
    scores = acc / jnp.maximum(cnt, 1).astype(F32)
    scores_row = jnp.transpose(scores)

    ii = jax.lax.broadcasted_iota(jnp.int32, (S0, S0), 0)
    jj = jax.lax.broadcasted_iota(jnp.int32, (S0, S0), 1)
    gt = (scores_row > scores).astype(jnp.int32)
    eq = ((scores_row == scores) & (jj < ii)).astype(jnp.int32)
    rank = jnp.sum(gt + eq, axis=1, keepdims=True)
    rank_row = jnp.transpose(rank)

    r_col = jax.lax.broadcasted_iota(jnp.int32, (PER_SPK, 1), 0)
    sel = (rank_row == r_col).astype(jnp.int32)
    p = jnp.sum(sel * s_row, axis=1, keepdims=True)
    conf = jnp.sum(sel.astype(F32) * scores_row, axis=1, keepdims=True)
    gate = jax.nn.sigmoid(2.0 * conf)

    t2 = jax.lax.broadcasted_iota(jnp.int32, (PER_SPK, T), 1)
    d2 = t2 - p
    win = (d2 >= -GATE_R) & (d2 <= GATE_R)
    df = d2.astype(F32) * (1.0 / SIGMA)
    w = jnp.where(win, jnp.exp(-0.5 * df * df) * a_row, 0.0)
    wsum = jnp.sum(w, axis=1, keepdims=True)
    wn = (w / (wsum + 1e-6)).astype(BF16)
    z = _dot(wn, h_ref[0].astype(BF16)).astype(BF16)

    k_seed = _dott(z, wkv_ref[0]) + bkv_ref[...]
    qk = jnp.tanh(_dott(k_seed.astype(BF16), wq) + bq).astype(BF16)
    return (_dott(qk, wqin) + bqin).astype(BF16), gate


def _prep_kernel(h0_ref, a0_ref, sp0_ref, wkv0_ref, bkv0_ref,
                 h1_ref, a1_ref, sp1_ref, wkv1_ref, bkv1_ref,
                 wq_ref, bq_ref, wqin_ref, bqin_ref, q_out, g_out):
    wq = wq_ref[...]
    bq = bq_ref[...]
    wqin = wqin_ref[...]
    bqin = bqin_ref[...]
    q0, g0 = _spk_track(h0_ref, a0_ref, sp0_ref, wkv0_ref, bkv0_ref,
                        wq, bq, wqin, bqin)
    q1, g1 = _spk_track(h1_ref, a1_ref, sp1_ref, wkv1_ref, bkv1_ref,
                        wq, bq, wqin, bqin)
    q_out[0, 0:PER_SPK] = q0
    q_out[0, PER_SPK:SQ] = q1
    g_out[0, 0:PER_SPK] = g0
    g_out[0, PER_SPK:SQ] = g1


def _prep(h0, a0, sp0, wkv0, bkv0, h1, a1, sp1, wkv1, bkv1, wq, bq, wqin,
          bqin):
    hspec = pl.BlockSpec((1, T, D_C), lambda b: (b, 0, 0))
    aspec = pl.BlockSpec((1, 1, T), lambda b: (b, 0, 0))
    sspec = pl.BlockSpec((1, 1, S0), lambda b: (b, 0, 0))
    wkvspec = pl.BlockSpec((1, D_MODEL, D_C), lambda b: (0, 0, 0))
    rowspec = pl.BlockSpec((1, D_MODEL), lambda b: (0, 0))
    wspec = pl.BlockSpec((D_MODEL, D_MODEL), lambda b: (0, 0))
    return pl.pallas_call(
        _prep_kernel,
        grid=(B,),
        in_specs=[hspec, aspec, sspec, wkvspec, rowspec,
                  hspec, aspec, sspec, wkvspec, rowspec,
                  wspec, rowspec, wspec, rowspec],
        out_specs=[
            pl.BlockSpec((1, SQ, D_MODEL), lambda b: (b, 0, 0)),
            pl.BlockSpec((1, SQ, 1), lambda b: (b, 0, 0)),
        ],
        out_shape=[
            jax.ShapeDtypeStruct((B, SQ, D_MODEL), BF16),
            jax.ShapeDtypeStruct((B, SQ, 1), F32),
        ],
    )(h0, a0, sp0, wkv0, bkv0, h1, a1, sp1, wkv1, bkv1, wq, bq, wqin, bqin)


def _wfuse_kernel(wk_ref, wv_ref, wmem_ref, bmem_ref, bv_ref,
                  wkf_out, wvf_out, bvf_out):
    wmem = wmem_ref[...]
    wkf_out[...] = _dot(wk_ref[...], wmem).astype(BF16)
    wvf_out[...] = _dot(wv_ref[...], wmem).astype(BF16)
    bvf_out[...] = _dott(bmem_ref[...].astype(BF16), wv_ref[...]) + bv_ref[...]


def _wfuse(wk, wv, wmem, bmem, bv):
    def full(shape):
        return pl.BlockSpec(shape, lambda: tuple(0 for _ in shape))
    return pl.pallas_call(
        _wfuse_kernel,
        in_specs=[full((D_MODEL, D_MODEL)), full((D_MODEL, D_MODEL)),
                  full((D_MODEL, D_PROJ)), full((1, D_MODEL)),
                  full((1, D_MODEL))],
        out_specs=[full((D_MODEL, D_PROJ)), full((D_MODEL, D_PROJ)),
                   full((1, D_MODEL))],
        out_shape=[
            jax.ShapeDtypeStruct((D_MODEL, D_PROJ), BF16),
            jax.ShapeDtypeStruct((D_MODEL, D_PROJ), BF16),
            jax.ShapeDtypeStruct((1, D_MODEL), F32),
        ],
    )(wk, wv, wmem, bmem, bv)


def _attn_kernel(pf_ref, wkf_ref, wvf_ref, bvf_ref, q_ref,
                 opw_ref, opb_ref, wo_ref, bo_ref, g_ref, a0_ref, a1_ref,
                 tags_ref, out_ref, m_s, l_s, acc_s, o_s, sc_s, pv_s):
    tb = pl.program_id(1)
    x = pf_ref[0].astype(BF16)
    kb = _dott(x, wkf_ref[...]).astype(BF16)
    vb = (_dott(x, wvf_ref[...]) + bvf_ref[...]).astype(BF16)
    qa = q_ref[0]

    for h in range(N_HEADS):
        qh = qa[:, h * HD:(h + 1) * HD]
        kh = kb[:, h * HD:(h + 1) * HD]
        sc_s[h] = _dott(qh, kh) * (1.0 / (HD ** 0.5))

    sc = sc_s[...]
    m_loc = jnp.max(sc, axis=2, keepdims=True)
    if NT > 1:
        m_old = jnp.where(tb == 0, jnp.full((N_HEADS, SQ, 1), -1e30, F32),
                          m_s[...])
        m_new = jnp.maximum(m_old, m_loc)
        resc = jnp.exp(m_old - m_new)
        m_s[...] = m_new
    else:
        m_new = m_loc
    e = jnp.exp(sc - m_new)
    lloc = jnp.sum(e, axis=2, keepdims=True)
    if NT > 1:
        l_old = jnp.where(tb == 0, jnp.zeros((N_HEADS, SQ, 1), F32), l_s[...])
        l_s[...] = l_old * resc + lloc
    eb = e.astype(BF16)

    for h in range(N_HEADS):
        pv_s[h] = _dot(eb[h], vb[:, h * HD:(h + 1) * HD])
    if NT > 1:
        acc_old = jnp.where(tb == 0, jnp.zeros((N_HEADS, SQ, HD), F32),
                            acc_s[...])
        acc_s[...] = acc_old * resc + pv_s[...]
    @pl.when(tb == NT - 1)
    def _():
        if NT > 1:
            onorm = acc_s[...] / l_s[...]
        else:
            onorm = pv_s[...] / lloc
        for h in range(N_HEADS):
            o_s[:, h * HD:(h + 1) * HD] = onorm[h]

    @pl.when(tb == NT - 1)
    def _():
        o = o_s[...]
        f = _dott(o, opw_ref[...]) + opb_ref[...]
        f = _dott(f, wo_ref[...]) + bo_ref[...]
        g = g_ref[0]
        a0 = a0_ref[0, :, 0:1]
        a1 = a1_ref[0, :, 0:1]
        den = a0 + a1 + 1e-6
        tags = tags_ref[...]
        slot = (a0 / den) * tags[0:1, :] + (a1 / den) * tags[1:2, :]
        out_ref[0] = f * g + slot


def _attn_out(pf, wkf, wvf, bvf, q_all, opw, opb, wo, bo, g_all, a0s,
              a1s, tags):
    stride = T // SQ
    wide = pl.BlockSpec((D_MODEL, D_PROJ), lambda b, t: (0, 0))
    row = pl.BlockSpec((1, D_MODEL), lambda b, t: (0, 0))
    sqd = pl.BlockSpec((1, SQ, D_MODEL), lambda b, t: (b, 0, 0))
    return pl.pallas_call(
        _attn_kernel,
        grid=(B, NT),
        in_specs=[
            pl.BlockSpec((1, TB, D_PROJ), lambda b, t: (b, t, 0)),
            wide, wide, row, sqd,
            pl.BlockSpec((D_MODEL, D_MODEL), lambda b, t: (0, 0)), row,
            pl.BlockSpec((D_MODEL, D_MODEL), lambda b, t: (0, 0)), row,
            pl.BlockSpec((1, SQ, 1), lambda b, t: (b, 0, 0)),
            pl.BlockSpec((1, SQ, stride), lambda b, t: (b, 0, 0)),
            pl.BlockSpec((1, SQ, stride), lambda b, t: (b, 0, 0)),
            pl.BlockSpec((2, D_MODEL), lambda b, t: (0, 0)),
        ],
        out_specs=sqd,
        out_shape=jax.ShapeDtypeStruct((B, SQ, D_MODEL), F32),
        scratch_shapes=[
            pltpu.VMEM((N_HEADS, SQ, 1), F32),
            pltpu.VMEM((N_HEADS, SQ, 1), F32),
            pltpu.VMEM((N_HEADS, SQ, HD), F32),
            pltpu.VMEM((SQ, D_MODEL), F32),
            pltpu.VMEM((N_HEADS, SQ, TB), F32),
            pltpu.VMEM((N_HEADS, SQ, HD), F32),
        ],
    )(pf, wkf, wvf, bvf, q_all, opw, opb, wo, bo, g_all, a0s, a1s, tags)


def kernel(proj_feats, h_ctc_0, h_ctc_1, A_0, A_1, spikes_0, spikes_1,
           W_mem, b_mem, W_kv_0, b_kv_0, W_kv_1, b_kv_1, W_q, b_q, W_o, b_o,
           in_proj_w, in_proj_b, out_proj_w, out_proj_b, tags):
    wqi = in_proj_w[0:D_MODEL].astype(BF16)
    wki = in_proj_w[D_MODEL:2 * D_MODEL].astype(BF16)
    wvi = in_proj_w[2 * D_MODEL:3 * D_MODEL].astype(BF16)
    bqi = in_proj_b[0:D_MODEL].reshape(1, D_MODEL)
    bvi = in_proj_b[2 * D_MODEL:3 * D_MODEL].reshape(1, D_MODEL)

    q_all, g_all = _prep(
        h_ctc_0, A_0.reshape(B, 1, T),
        spikes_0.reshape(B, 1, S0),
        W_kv_0[:D_MODEL].reshape(1, D_MODEL, D_C).astype(BF16),
        b_kv_0[:D_MODEL].reshape(1, D_MODEL),
        h_ctc_1, A_1.reshape(B, 1, T),
        spikes_1.reshape(B, 1, S0),
        W_kv_1[:D_MODEL].reshape(1, D_MODEL, D_C).astype(BF16),
        b_kv_1[:D_MODEL].reshape(1, D_MODEL),
        W_q.astype(BF16), b_q.reshape(1, D_MODEL), wqi, bqi)

    wkf, wvf, bvf = _wfuse(wki, wvi, W_mem.astype(BF16),
                           b_mem.reshape(1, D_MODEL), bvi)

    a0s = A_0.reshape(B, SQ, T // SQ)
    a1s = A_1.reshape(B, SQ, T // SQ)
    return _attn_out(proj_feats, wkf, wvf, bvf, q_all,
                     out_proj_w, out_proj_b.reshape(1, D_MODEL), W_o,
                     b_o.reshape(1, D_MODEL), g_all, a0s, a1s, tags)
